# baseline (speedup 1.0000x reference)
import functools

import jax
import jax.numpy as jnp
from jax.experimental import pallas as pl
from jax.experimental.pallas import tpu as pltpu


def _fwd_kernel(mb_ref, seq_ref, px_ref, py_ref, lens_ref, out_ref,
                emit_ref, offs_ref, *, num_b, seq_len, chunk):
    py = py_ref[...]
    l1mpy = jnp.log1p(-py)
    w = (jnp.log(py) - l1mpy).astype(jnp.bfloat16)
    bias = jnp.sum(l1mpy, axis=1).reshape(1, -1)
    t_iota = jax.lax.broadcasted_iota(jnp.int32, (seq_len, 1), 0)

    for b in range(num_b):
        s = seq_ref[mb_ref[b]].astype(jnp.bfloat16)
        e = jax.lax.dot_general(s, w, (((1,), (1,)), ((), ())),
                                preferred_element_type=jnp.float32) + bias
        me = jnp.max(e, axis=1, keepdims=True)
        emit_ref[b] = jnp.exp(e - me)
        offs_ref[pl.ds(b, 1), :] = jnp.sum(
            jnp.where(t_iota < lens_ref[b, 0], me, 0.0), axis=0,
            keepdims=True)

    px = px_ref[...].astype(jnp.bfloat16)
    lens = lens_ref[...]

    p = px_ref[0:1, :] * emit_ref[:, 0, :]
    macc = jnp.zeros_like(offs_ref)
    cap_p = jnp.where(lens == 1, p, 0.0)
    cap_m = jnp.zeros_like(macc)

    def steps(k0, blk, p, macc, cap_p, cap_m, js):
        for j in js:
            t = k0 + j
            q = jax.lax.dot_general(p.astype(jnp.bfloat16), px,
                                    (((1,), (0,)), ((), ())),
                                    preferred_element_type=jnp.float32)
            p = q * blk[:, j, :]
            hit = lens == t + 1
            cap_p = jnp.where(hit, p, cap_p)
        in_range = (lens > k0) & (lens <= k0 + chunk)
        cap_m = jnp.where(in_range, macc, cap_m)
        mm = jnp.max(p, axis=1, keepdims=True)
        p = p * (1.0 / mm)
        macc = macc + jnp.log(mm)
        return p, macc, cap_p, cap_m

    p, macc, cap_p, cap_m = steps(
        0, emit_ref[:, 0:chunk, :], p, macc, cap_p, cap_m,
        range(1, chunk))

    def chunk_body(k, carry):
        blk = emit_ref[:, pl.ds(k * chunk, chunk), :]
        return steps(k * chunk, blk, *carry, range(chunk))

    p, macc, cap_p, cap_m = jax.lax.fori_loop(
        1, seq_len // chunk, chunk_body, (p, macc, cap_p, cap_m))

    ll = cap_m + offs_ref[...] + jnp.log(
        jnp.sum(cap_p, axis=1, keepdims=True))
    out_ref[...] = jnp.sum(ll, axis=0, keepdims=True)


def kernel(sequences, lengths, mb, probs_x, probs_y, scale=1.0):
    num_seq, seq_len, data_dim = sequences.shape
    hidden = probs_x.shape[0]
    num_b = mb.shape[0]
    chunk = 8

    lens = lengths[mb].reshape(num_b, 1)

    grid_spec = pltpu.PrefetchScalarGridSpec(
        num_scalar_prefetch=1,
        grid=(1,),
        in_specs=[
            pl.BlockSpec((num_seq, seq_len, data_dim),
                         lambda i, mb_ref: (0, 0, 0)),
            pl.BlockSpec((hidden, hidden), lambda i, mb_ref: (0, 0)),
            pl.BlockSpec((hidden, data_dim), lambda i, mb_ref: (0, 0)),
            pl.BlockSpec((num_b, 1), lambda i, mb_ref: (0, 0)),
        ],
        out_specs=pl.BlockSpec((1, 1), lambda i, mb_ref: (0, 0)),
        scratch_shapes=[
            pltpu.VMEM((num_b, seq_len, hidden), jnp.float32),
            pltpu.VMEM((num_b, 1), jnp.float32),
        ],
    )

    out = pl.pallas_call(
        functools.partial(_fwd_kernel, num_b=num_b, seq_len=seq_len, chunk=chunk),
        grid_spec=grid_spec,
        out_shape=jax.ShapeDtypeStruct((1, 1), jnp.float32),
    )(mb, sequences, probs_x, probs_y, lens)

    return (scale * out[0, 0]).astype(jnp.float32)


# speedup vs baseline: 1.0691x; 1.0691x over previous
import functools

import jax
import jax.numpy as jnp
from jax.experimental import pallas as pl
from jax.experimental.pallas import tpu as pltpu


def _fwd_kernel(mb_ref, seq_ref, px_ref, py_ref, lens_ref, out_ref,
                emit_ref, offs_ref, *, num_b, seq_len, chunk):
    py = py_ref[...]
    l1mpy = jnp.log1p(-py)
    w = (jnp.log(py) - l1mpy).astype(jnp.bfloat16)
    bias = jnp.sum(l1mpy, axis=1).reshape(1, -1)
    t_iota = jax.lax.broadcasted_iota(jnp.int32, (seq_len, 1), 0)

    for b in range(num_b):
        s = seq_ref[mb_ref[b]].astype(jnp.bfloat16)
        e = jax.lax.dot_general(s, w, (((1,), (1,)), ((), ())),
                                preferred_element_type=jnp.float32) + bias
        me = jnp.max(e, axis=1, keepdims=True)
        emit_ref[b] = jnp.exp(e - me)
        offs_ref[pl.ds(b, 1), :] = jnp.sum(
            jnp.where(t_iota < lens_ref[b, 0], me, 0.0), axis=0,
            keepdims=True)

    px = px_ref[...].astype(jnp.bfloat16)
    lens = lens_ref[...]
    half = seq_len // 2
    nck = half // chunk

    p = px_ref[0:1, :] * emit_ref[:, 0, :]
    macc = jnp.zeros_like(offs_ref)
    cap_p = jnp.where(lens == 1, p, 0.0)
    cap_m = jnp.zeros_like(macc)
    c = jnp.ones((lens.shape[0], px.shape[0]), jnp.float32)
    bacc = jnp.zeros_like(macc)

    def fwd_steps(k0, blk, p, macc, cap_p, cap_m, js):
        for j in js:
            t = k0 + j
            q = jax.lax.dot_general(p.astype(jnp.bfloat16), px,
                                    (((1,), (0,)), ((), ())),
                                    preferred_element_type=jnp.float32)
            p = q * blk[:, j, :]
            cap_p = jnp.where(lens == t + 1, p, cap_p)
        in_range = (lens > k0) & (lens <= k0 + chunk)
        cap_m = jnp.where(in_range, macc, cap_m)
        mm = jnp.max(p, axis=1, keepdims=True)
        p = p * (1.0 / mm)
        macc = macc + jnp.log(mm)
        return p, macc, cap_p, cap_m

    def bwd_steps(kb, blk, c, bacc):
        for j in reversed(range(chunk)):
            t = kb * chunk + j
            w = (c * blk[:, j, :]).astype(jnp.bfloat16)
            c = jax.lax.dot_general(w, px, (((1,), (1,)), ((), ())),
                                    preferred_element_type=jnp.float32)
            c = jnp.where(lens == t, 1.0, c)
        injected = (lens >= kb * chunk) & (lens <= kb * chunk + chunk)
        bacc = jnp.where(injected, 0.0, bacc)
        mm = jnp.max(c, axis=1, keepdims=True)
        c = c * (1.0 / mm)
        bacc = bacc + jnp.log(mm)
        return c, bacc

    p, macc, cap_p, cap_m = fwd_steps(
        0, emit_ref[:, 0:chunk, :], p, macc, cap_p, cap_m, range(1, chunk))
    c, bacc = bwd_steps(2 * nck - 1,
                        emit_ref[:, (2 * nck - 1) * chunk:, :], c, bacc)

    def chunk_body(k, carry):
        p, macc, cap_p, cap_m, c, bacc = carry
        blkf = emit_ref[:, pl.ds(k * chunk, chunk), :]
        kb = 2 * nck - 1 - k
        blkb = emit_ref[:, pl.ds(kb * chunk, chunk), :]
        p, macc, cap_p, cap_m = fwd_steps(k * chunk, blkf, p, macc,
                                          cap_p, cap_m, range(chunk))
        c, bacc = bwd_steps(kb, blkb, c, bacc)
        return p, macc, cap_p, cap_m, c, bacc

    p, macc, cap_p, cap_m, c, bacc = jax.lax.fori_loop(
        1, nck, chunk_body, (p, macc, cap_p, cap_m, c, bacc))

    ll_f = cap_m + jnp.log(jnp.sum(cap_p, axis=1, keepdims=True))
    ll_b = macc + bacc + jnp.log(
        jnp.sum(p * c, axis=1, keepdims=True))
    ll = offs_ref[...] + jnp.where(lens <= half, ll_f, ll_b)
    out_ref[...] = jnp.sum(ll, axis=0, keepdims=True)


def kernel(sequences, lengths, mb, probs_x, probs_y, scale=1.0):
    num_seq, seq_len, data_dim = sequences.shape
    hidden = probs_x.shape[0]
    num_b = mb.shape[0]
    chunk = 8

    lens = lengths[mb].reshape(num_b, 1)

    grid_spec = pltpu.PrefetchScalarGridSpec(
        num_scalar_prefetch=1,
        grid=(1,),
        in_specs=[
            pl.BlockSpec((num_seq, seq_len, data_dim),
                         lambda i, mb_ref: (0, 0, 0)),
            pl.BlockSpec((hidden, hidden), lambda i, mb_ref: (0, 0)),
            pl.BlockSpec((hidden, data_dim), lambda i, mb_ref: (0, 0)),
            pl.BlockSpec((num_b, 1), lambda i, mb_ref: (0, 0)),
        ],
        out_specs=pl.BlockSpec((1, 1), lambda i, mb_ref: (0, 0)),
        scratch_shapes=[
            pltpu.VMEM((num_b, seq_len, hidden), jnp.float32),
            pltpu.VMEM((num_b, 1), jnp.float32),
        ],
    )

    out = pl.pallas_call(
        functools.partial(_fwd_kernel, num_b=num_b, seq_len=seq_len, chunk=chunk),
        grid_spec=grid_spec,
        out_shape=jax.ShapeDtypeStruct((1, 1), jnp.float32),
    )(mb, sequences, probs_x, probs_y, lens)

    return (scale * out[0, 0]).astype(jnp.float32)


# speedup vs baseline: 1.2922x; 1.2086x over previous
import functools

import jax
import jax.numpy as jnp
from jax.experimental import pallas as pl
from jax.experimental.pallas import tpu as pltpu


def _fwd_kernel(mb_ref, seq_ref, px_ref, py_ref, lens_ref, out_ref,
                emit_ref, offs_ref, *, num_b, seq_len, chunk):
    py = py_ref[...]
    l1mpy = jnp.log1p(-py)
    w = (jnp.log(py) - l1mpy).astype(jnp.bfloat16)
    bias = jnp.sum(l1mpy, axis=1).reshape(1, -1)
    t_iota = jax.lax.broadcasted_iota(jnp.int32, (seq_len, 1), 0)

    for b in range(num_b):
        s = seq_ref[mb_ref[b]].astype(jnp.bfloat16)
        e = jax.lax.dot_general(s, w, (((1,), (1,)), ((), ())),
                                preferred_element_type=jnp.float32) + bias
        me = jnp.max(e, axis=1, keepdims=True)
        emit_ref[b] = jnp.exp(e - me)
        offs_ref[pl.ds(b, 1), :] = jnp.sum(
            jnp.where(t_iota < lens_ref[b, 0], me, 0.0), axis=0,
            keepdims=True)

    px = px_ref[...].astype(jnp.bfloat16)
    lens = lens_ref[...]
    half = seq_len // 2
    nck = half // chunk

    p = px_ref[0:1, :] * emit_ref[:, 0, :]
    macc = jnp.zeros_like(offs_ref)
    cap_p = jnp.where(lens == 1, p, 0.0)
    cap_m = jnp.zeros_like(macc)
    c = jnp.ones((lens.shape[0], px.shape[0]), jnp.float32)
    bacc = jnp.zeros_like(macc)

    def fwd_steps(k0, blk, p, macc, cap_p, cap_m, js):
        for j in js:
            t = k0 + j
            q = jax.lax.dot_general(p.astype(jnp.bfloat16), px,
                                    (((1,), (0,)), ((), ())),
                                    preferred_element_type=jnp.float32)
            p = q * blk[:, j, :]
            cap_p = jnp.where(lens == t + 1, p, cap_p)
        in_range = (lens > k0) & (lens <= k0 + chunk)
        cap_m = jnp.where(in_range, macc, cap_m)
        mm = jnp.max(p, axis=1, keepdims=True)
        p = p * (1.0 / mm)
        macc = macc + jnp.log(mm)
        return p, macc, cap_p, cap_m

    def bwd_steps(kb, blk, c, bacc):
        for j in reversed(range(chunk)):
            t = kb * chunk + j
            w = (c * blk[:, j, :]).astype(jnp.bfloat16)
            c = jax.lax.dot_general(w, px, (((1,), (1,)), ((), ())),
                                    preferred_element_type=jnp.float32)
            c = jnp.where(lens == t, 1.0, c)
        injected = (lens >= kb * chunk) & (lens <= kb * chunk + chunk)
        bacc = jnp.where(injected, 0.0, bacc)
        mm = jnp.max(c, axis=1, keepdims=True)
        c = c * (1.0 / mm)
        bacc = bacc + jnp.log(mm)
        return c, bacc

    p, macc, cap_p, cap_m = fwd_steps(
        0, emit_ref[:, 0:chunk, :], p, macc, cap_p, cap_m, range(1, chunk))
    c, bacc = bwd_steps(2 * nck - 1,
                        emit_ref[:, (2 * nck - 1) * chunk:, :], c, bacc)

    def chunk_body(k, carry):
        p, macc, cap_p, cap_m, c, bacc = carry
        blkf = emit_ref[:, pl.ds(k * chunk, chunk), :]
        kb = 2 * nck - 1 - k
        blkb = emit_ref[:, pl.ds(kb * chunk, chunk), :]
        p, macc, cap_p, cap_m = fwd_steps(k * chunk, blkf, p, macc,
                                          cap_p, cap_m, range(chunk))
        c, bacc = bwd_steps(kb, blkb, c, bacc)
        return p, macc, cap_p, cap_m, c, bacc

    p, macc, cap_p, cap_m, c, bacc = jax.lax.fori_loop(
        1, nck, chunk_body, (p, macc, cap_p, cap_m, c, bacc))

    ll_f = cap_m + jnp.log(jnp.sum(cap_p, axis=1, keepdims=True))
    ll_b = macc + bacc + jnp.log(
        jnp.sum(p * c, axis=1, keepdims=True))
    ll = offs_ref[...] + jnp.where(lens <= half, ll_f, ll_b)
    out_ref[...] = jnp.sum(ll, axis=0, keepdims=True)


def kernel(sequences, lengths, mb, probs_x, probs_y, scale=1.0):
    num_seq, seq_len, data_dim = sequences.shape
    hidden = probs_x.shape[0]
    num_b = mb.shape[0]
    chunk = 8

    lens = lengths[mb].reshape(num_b, 1)

    grid_spec = pltpu.PrefetchScalarGridSpec(
        num_scalar_prefetch=1,
        grid=(1,),
        in_specs=[
            pl.BlockSpec((num_seq, seq_len, data_dim),
                         lambda i, mb_ref: (0, 0, 0)),
            pl.BlockSpec((hidden, hidden), lambda i, mb_ref: (0, 0)),
            pl.BlockSpec((hidden, data_dim), lambda i, mb_ref: (0, 0)),
            pl.BlockSpec((num_b, 1), lambda i, mb_ref: (0, 0)),
        ],
        out_specs=pl.BlockSpec((1, 1), lambda i, mb_ref: (0, 0)),
        scratch_shapes=[
            pltpu.VMEM((num_b, seq_len, hidden), jnp.float32),
            pltpu.VMEM((num_b, 1), jnp.float32),
        ],
    )

    out = pl.pallas_call(
        functools.partial(_fwd_kernel, num_b=num_b, seq_len=seq_len, chunk=chunk),
        grid_spec=grid_spec,
        out_shape=jax.ShapeDtypeStruct((1, 1), jnp.float32),
    )(mb, sequences.astype(jnp.int8), probs_x, probs_y, lens)

    return (scale * out[0, 0]).astype(jnp.float32)


# speedup vs baseline: 1.2962x; 1.0031x over previous
import functools

import jax
import jax.numpy as jnp
from jax.experimental import pallas as pl
from jax.experimental.pallas import tpu as pltpu


def _fwd_kernel(mb_ref, seq_ref, px_ref, py_ref, lens_ref, out_ref,
                emit_ref, offs_ref, *, num_b, seq_len, chunk):
    py = py_ref[...]
    l1mpy = jnp.log1p(-py)
    w = (jnp.log(py) - l1mpy).astype(jnp.bfloat16)
    bias = jnp.sum(l1mpy, axis=1).reshape(1, -1)
    t_iota = jax.lax.broadcasted_iota(jnp.int32, (seq_len, 1), 0)

    for b in range(num_b):
        s = seq_ref[mb_ref[b]].astype(jnp.bfloat16)
        e = jax.lax.dot_general(s, w, (((1,), (1,)), ((), ())),
                                preferred_element_type=jnp.float32) + bias
        me = jnp.max(e, axis=1, keepdims=True)
        emit_ref[b] = jnp.exp(e - me).astype(jnp.bfloat16)
        offs_ref[pl.ds(b, 1), :] = jnp.sum(
            jnp.where(t_iota < lens_ref[b, 0], me, 0.0), axis=0,
            keepdims=True)

    px = px_ref[...].astype(jnp.bfloat16)
    lens = lens_ref[...]
    half = seq_len // 2
    nck = half // chunk

    p = (px[0:1, :] * emit_ref[:, 0, :]).astype(jnp.float32)
    macc = jnp.zeros_like(offs_ref)
    cap_p = jnp.where(lens == 1, p, 0.0)
    cap_m = jnp.zeros_like(macc)
    c = jnp.ones((lens.shape[0], px.shape[0]), jnp.float32)
    bacc = jnp.zeros_like(macc)

    def fwd_steps(k0, blk, p, macc, cap_p, cap_m, js):
        for j in js:
            t = k0 + j
            q = jax.lax.dot_general(p.astype(jnp.bfloat16), px,
                                    (((1,), (0,)), ((), ())),
                                    preferred_element_type=jnp.float32)
            p = q * blk[:, j, :].astype(jnp.float32)
            cap_p = jnp.where(lens == t + 1, p, cap_p)
        in_range = (lens > k0) & (lens <= k0 + chunk)
        cap_m = jnp.where(in_range, macc, cap_m)
        mm = jnp.max(p, axis=1, keepdims=True)
        p = p * (1.0 / mm)
        macc = macc + jnp.log(mm)
        return p, macc, cap_p, cap_m

    def bwd_steps(kb, blk, c, bacc):
        for j in reversed(range(chunk)):
            t = kb * chunk + j
            w = (c * blk[:, j, :].astype(jnp.float32)).astype(jnp.bfloat16)
            c = jax.lax.dot_general(w, px, (((1,), (1,)), ((), ())),
                                    preferred_element_type=jnp.float32)
            c = jnp.where(lens == t, 1.0, c)
        injected = (lens >= kb * chunk) & (lens <= kb * chunk + chunk)
        bacc = jnp.where(injected, 0.0, bacc)
        mm = jnp.max(c, axis=1, keepdims=True)
        c = c * (1.0 / mm)
        bacc = bacc + jnp.log(mm)
        return c, bacc

    p, macc, cap_p, cap_m = fwd_steps(
        0, emit_ref[:, 0:chunk, :], p, macc, cap_p, cap_m, range(1, chunk))
    c, bacc = bwd_steps(2 * nck - 1,
                        emit_ref[:, (2 * nck - 1) * chunk:, :], c, bacc)

    def chunk_body(k, carry):
        p, macc, cap_p, cap_m, c, bacc = carry
        blkf = emit_ref[:, pl.ds(k * chunk, chunk), :]
        kb = 2 * nck - 1 - k
        blkb = emit_ref[:, pl.ds(kb * chunk, chunk), :]
        p, macc, cap_p, cap_m = fwd_steps(k * chunk, blkf, p, macc,
                                          cap_p, cap_m, range(chunk))
        c, bacc = bwd_steps(kb, blkb, c, bacc)
        return p, macc, cap_p, cap_m, c, bacc

    p, macc, cap_p, cap_m, c, bacc = jax.lax.fori_loop(
        1, nck, chunk_body, (p, macc, cap_p, cap_m, c, bacc))

    ll_f = cap_m + jnp.log(jnp.sum(cap_p, axis=1, keepdims=True))
    ll_b = macc + bacc + jnp.log(
        jnp.sum(p * c, axis=1, keepdims=True))
    ll = offs_ref[...] + jnp.where(lens <= half, ll_f, ll_b)
    out_ref[...] = jnp.sum(ll, axis=0, keepdims=True)


def kernel(sequences, lengths, mb, probs_x, probs_y, scale=1.0):
    num_seq, seq_len, data_dim = sequences.shape
    hidden = probs_x.shape[0]
    num_b = mb.shape[0]
    chunk = 8

    lens = lengths[mb].reshape(num_b, 1)

    grid_spec = pltpu.PrefetchScalarGridSpec(
        num_scalar_prefetch=1,
        grid=(1,),
        in_specs=[
            pl.BlockSpec((num_seq, seq_len, data_dim),
                         lambda i, mb_ref: (0, 0, 0)),
            pl.BlockSpec((hidden, hidden), lambda i, mb_ref: (0, 0)),
            pl.BlockSpec((hidden, data_dim), lambda i, mb_ref: (0, 0)),
            pl.BlockSpec((num_b, 1), lambda i, mb_ref: (0, 0)),
        ],
        out_specs=pl.BlockSpec((1, 1), lambda i, mb_ref: (0, 0)),
        scratch_shapes=[
            pltpu.VMEM((num_b, seq_len, hidden), jnp.bfloat16),
            pltpu.VMEM((num_b, 1), jnp.float32),
        ],
    )

    out = pl.pallas_call(
        functools.partial(_fwd_kernel, num_b=num_b, seq_len=seq_len, chunk=chunk),
        grid_spec=grid_spec,
        out_shape=jax.ShapeDtypeStruct((1, 1), jnp.float32),
    )(mb, sequences.astype(jnp.int8), probs_x, probs_y, lens)

    return (scale * out[0, 0]).astype(jnp.float32)
